# Initial kernel scaffold; baseline (speedup 1.0000x reference)
import jax
import jax.numpy as jnp
from jax.experimental import pallas as pl


def kernel(features, edge_index, aspect_indices, W1, b1, W2, b2, Wc, bc):
    raise NotImplementedError("write your pallas kernel here")


# speedup vs baseline: 8.1841x; 8.1841x over previous
import functools

import jax
import jax.numpy as jnp
from jax import lax
from jax.experimental import pallas as pl
from jax.experimental.pallas import tpu as pltpu
from jax.experimental.pallas import tpu_sc as plsc

N = 10000
NP = 10240
E = 320000
EP = 327680
D = 128
DH = 64
A = 1024
NC = 2
NS = 16
NW = NC * NS
ET = EP // NW
CH = 64
NCH = ET // CH
RPT = NP // NS
L = 16

_f32 = jnp.float32
_i32 = jnp.int32
_mesh = plsc.VectorSubcoreMesh(core_axis_name="c", subcore_axis_name="s")
_sc_params = pltpu.CompilerParams(use_tc_tiling_on_sc=False)
_MASK = (1 << 14) - 1


def _unpack(pki, srci, dsti, want_src):
    def body(k, carry):
        for j in range(CH // L):
            v = pki[k, pl.ds(j * L, L)]
            if want_src:
                srci[k, pl.ds(j * L, L)] = v & _MASK
            dsti[k, pl.ds(j * L, L)] = lax.shift_right_logical(v, 14)
        return carry

    lax.fori_loop(0, NCH, body, 0)


@functools.partial(
    pl.kernel,
    out_type=(jax.ShapeDtypeStruct((NP, 16), _f32),
              jax.ShapeDtypeStruct((NP, 16), _f32)),
    mesh=_mesh,
    compiler_params=_sc_params,
    scratch_types=[
        pltpu.VMEM((NCH, CH), _i32),
        pltpu.VMEM((NCH, CH), _i32),
        pltpu.VMEM((CH, 16), _f32),
        pltpu.VMEM_SHARED((NP, 16), _f32),
    ],
)
def _deg_kernel(pk2, ones16, out0, out1, pki, dsti, ones_v, acc):
    c = lax.axis_index("c")
    s = lax.axis_index("s")
    wid = s * NC + c
    r0 = s * RPT
    pltpu.sync_copy(pk2.at[pl.ds(wid * NCH, NCH)], pki)
    pltpu.sync_copy(ones16.at[pl.ds(0, CH)], ones_v)
    pltpu.sync_copy(ones16.at[pl.ds(r0, RPT)], acc.at[pl.ds(r0, RPT)])
    _unpack(pki, None, dsti, want_src=False)
    plsc.subcore_barrier()

    def body(k, carry):
        pltpu.sync_copy(ones_v, acc.at[dsti.at[k]], add=True)
        return carry

    lax.fori_loop(0, NCH, body, 0)
    plsc.subcore_barrier()

    @pl.when(c == 0)
    def _():
        pltpu.sync_copy(acc.at[pl.ds(r0, RPT)], out0.at[pl.ds(r0, RPT)])

    @pl.when(c == 1)
    def _():
        pltpu.sync_copy(acc.at[pl.ds(r0, RPT)], out1.at[pl.ds(r0, RPT)])


@functools.partial(
    pl.kernel,
    out_type=tuple(jax.ShapeDtypeStruct((NP, DH), _f32) for _ in range(4)),
    mesh=_mesh,
    compiler_params=_sc_params,
    scratch_types=[
        pltpu.VMEM((NCH, CH), _i32),
        pltpu.VMEM((NCH, CH), _i32),
        pltpu.VMEM((NCH, CH), _i32),
        pltpu.VMEM((CH, DH), _f32),
        pltpu.VMEM((CH, DH), _f32),
        pltpu.VMEM_SHARED((NP, DH), _f32),
        pltpu.SemaphoreType.DMA,
        pltpu.SemaphoreType.DMA,
    ],
)
def _edge_kernel(ylo, yhi, pk2, zc, o0lo, o0hi, o1lo, o1hi,
                 pki, srci, dsti, rows_a, rows_b, acc, sem_a, sem_b):
    c = lax.axis_index("c")
    s = lax.axis_index("s")
    wid = s * NC + c
    r0 = s * RPT
    pltpu.sync_copy(pk2.at[pl.ds(wid * NCH, NCH)], pki)
    _unpack(pki, srci, dsti, want_src=True)

    for y, out0, out1 in ((ylo, o0lo, o1lo), (yhi, o0hi, o1hi)):
        pltpu.sync_copy(zc.at[pl.ds(r0, RPT)], acc.at[pl.ds(r0, RPT)])
        plsc.subcore_barrier()

        pltpu.async_copy(y.at[srci.at[0]], rows_a, sem_a)

        def body(i, carry, y=y):
            k = 2 * i
            pltpu.async_copy(y.at[srci.at[k + 1]], rows_b, sem_b)
            pltpu.make_async_copy(y.at[srci.at[k]], rows_a, sem_a).wait()
            pltpu.sync_copy(rows_a, acc.at[dsti.at[k]], add=True)

            @pl.when(k + 2 < NCH)
            def _():
                pltpu.async_copy(y.at[srci.at[k + 2]], rows_a, sem_a)

            pltpu.make_async_copy(y.at[srci.at[k + 1]], rows_b, sem_b).wait()
            pltpu.sync_copy(rows_b, acc.at[dsti.at[k + 1]], add=True)
            return carry

        lax.fori_loop(0, NCH // 2, body, 0)
        plsc.subcore_barrier()

        @pl.when(c == 0)
        def _():
            pltpu.sync_copy(acc.at[pl.ds(r0, RPT)], out0.at[pl.ds(r0, RPT)])

        @pl.when(c == 1)
        def _():
            pltpu.sync_copy(acc.at[pl.ds(r0, RPT)], out1.at[pl.ds(r0, RPT)])

        plsc.subcore_barrier()


_APT = A // NW


@functools.partial(
    pl.kernel,
    out_type=tuple(jax.ShapeDtypeStruct((A, DH), _f32) for _ in range(6))
    + (jax.ShapeDtypeStruct((A, 16), _f32),),
    mesh=_mesh,
    compiler_params=_sc_params,
    scratch_types=[
        pltpu.VMEM((_APT,), _i32),
        pltpu.VMEM((_APT, DH), _f32),
        pltpu.VMEM((_APT, 16), _f32),
        pltpu.SemaphoreType.DMA,
    ],
)
def _gather_kernel(t0, t1, t2, t3, t4, t5, di16, asp,
                   g0, g1, g2, g3, g4, g5, gd,
                   idxv, rbuf, rbuf16, sem):
    c = lax.axis_index("c")
    s = lax.axis_index("s")
    wid = s * NC + c
    base = wid * _APT
    pltpu.sync_copy(asp.at[pl.ds(base, _APT)], idxv)

    for tab, g in ((t0, g0), (t1, g1), (t2, g2), (t3, g3), (t4, g4), (t5, g5)):
        pltpu.async_copy(tab.at[idxv], rbuf, sem).wait()
        pltpu.sync_copy(rbuf, g.at[pl.ds(base, _APT)])

    pltpu.async_copy(di16.at[idxv], rbuf16, sem).wait()
    pltpu.sync_copy(rbuf16, gd.at[pl.ds(base, _APT)])


_RB = 1024


def _tc1_body(d0, d1, x, w, ylo_ref, yhi_ref, di_ref, di16_ref):
    deg = d0[...] + d1[...] - 1.0
    di = jnp.where(deg > 0, lax.rsqrt(jnp.maximum(deg, 1e-12)), 0.0)
    y = jnp.dot(x[...], w[...], preferred_element_type=_f32) * di
    ylo_ref[...] = y[:, :DH]
    yhi_ref[...] = y[:, DH:]
    di_ref[...] = di
    di16_ref[...] = jnp.broadcast_to(di, (di.shape[0], 16))


def _tc1(d0s, d1s, x, w):
    return pl.pallas_call(
        _tc1_body,
        grid=(NP // _RB,),
        in_specs=[
            pl.BlockSpec((_RB, 1), lambda i: (i, 0)),
            pl.BlockSpec((_RB, 1), lambda i: (i, 0)),
            pl.BlockSpec((_RB, D), lambda i: (i, 0)),
            pl.BlockSpec((D, D), lambda i: (0, 0)),
        ],
        out_specs=[
            pl.BlockSpec((_RB, DH), lambda i: (i, 0)),
            pl.BlockSpec((_RB, DH), lambda i: (i, 0)),
            pl.BlockSpec((_RB, 1), lambda i: (i, 0)),
            pl.BlockSpec((_RB, 16), lambda i: (i, 0)),
        ],
        out_shape=[
            jax.ShapeDtypeStruct((NP, DH), _f32),
            jax.ShapeDtypeStruct((NP, DH), _f32),
            jax.ShapeDtypeStruct((NP, 1), _f32),
            jax.ShapeDtypeStruct((NP, 16), _f32),
        ],
    )(d0s, d1s, x, w)


def _tc2_body(a0lo, a0hi, a1lo, a1hi, ylo, yhi, di, b1, w2,
              y2lo_ref, y2hi_ref):
    dv = di[...]
    hlo = a0lo[...] + a1lo[...] + ylo[...]
    hhi = a0hi[...] + a1hi[...] + yhi[...]
    h = dv * jnp.concatenate([hlo, hhi], axis=1) + b1[...]
    h = jnp.maximum(h, 0.0)
    y2 = jnp.dot(h, w2[...], preferred_element_type=_f32) * dv
    y2lo_ref[...] = y2[:, :DH]
    y2hi_ref[...] = y2[:, DH:]


def _tc2(a0lo, a0hi, a1lo, a1hi, ylo, yhi, di, b1r, w2):
    blk = pl.BlockSpec((_RB, DH), lambda i: (i, 0))
    return pl.pallas_call(
        _tc2_body,
        grid=(NP // _RB,),
        in_specs=[
            blk, blk, blk, blk, blk, blk,
            pl.BlockSpec((_RB, 1), lambda i: (i, 0)),
            pl.BlockSpec((1, D), lambda i: (0, 0)),
            pl.BlockSpec((D, D), lambda i: (0, 0)),
        ],
        out_specs=[blk, blk],
        out_shape=[
            jax.ShapeDtypeStruct((NP, DH), _f32),
            jax.ShapeDtypeStruct((NP, DH), _f32),
        ],
    )(a0lo, a0hi, a1lo, a1hi, ylo, yhi, di, b1r, w2)


def _tc3_body(g0, g1, g2, g3, g4, g5, di, b2, wc, bc, out_ref):
    hlo = g0[...] + g2[...] + g4[...]
    hhi = g1[...] + g3[...] + g5[...]
    h = di[...] * jnp.concatenate([hlo, hhi], axis=1) + b2[...]
    out_ref[...] = jnp.dot(h, wc[...], preferred_element_type=_f32) + bc[...]


def _tc3(gs, gd, b2r, wcp, bcp):
    blk = pl.BlockSpec((A, DH), lambda i: (0, 0))
    return pl.pallas_call(
        _tc3_body,
        grid=(1,),
        in_specs=[
            blk, blk, blk, blk, blk, blk,
            pl.BlockSpec((A, 1), lambda i: (0, 0)),
            pl.BlockSpec((1, D), lambda i: (0, 0)),
            pl.BlockSpec((D, D), lambda i: (0, 0)),
            pl.BlockSpec((1, D), lambda i: (0, 0)),
        ],
        out_specs=pl.BlockSpec((A, D), lambda i: (0, 0)),
        out_shape=jax.ShapeDtypeStruct((A, D), _f32),
    )(*gs, gd, b2r, wcp, bcp)


@jax.jit
def kernel(features, edge_index, aspect_indices, W1, b1, W2, b2, Wc, bc):
    pad_src = jnp.zeros((EP - E,), edge_index.dtype)
    pad_dst = jnp.full((EP - E,), N, edge_index.dtype)
    src = jnp.concatenate([edge_index[0], pad_src])
    dst = jnp.concatenate([edge_index[1], pad_dst])
    pk2 = (src | (dst << 14)).reshape(EP // CH, CH)
    xp = jnp.pad(features, ((0, NP - N), (0, 0)))
    ones16 = jnp.ones((NP, 16), _f32)
    zc = jnp.zeros((NP, DH), _f32)
    b1r = b1.reshape(1, D)
    b2r = b2.reshape(1, D)
    wcp = jnp.pad(Wc, ((0, 0), (0, D - Wc.shape[1])))
    bcp = jnp.pad(bc, (0, D - bc.shape[0])).reshape(1, D)

    d0, d1 = _deg_kernel(pk2, ones16)
    y1lo, y1hi, dinv, di16 = _tc1(d0[:, :1], d1[:, :1], xp, W1)
    a0lo, a0hi, a1lo, a1hi = _edge_kernel(y1lo, y1hi, pk2, zc)
    y2lo, y2hi = _tc2(a0lo, a0hi, a1lo, a1hi, y1lo, y1hi, dinv, b1r, W2)
    c0lo, c0hi, c1lo, c1hi = _edge_kernel(y2lo, y2hi, pk2, zc)
    gs = _gather_kernel(c0lo, c0hi, c1lo, c1hi, y2lo, y2hi,
                        di16, aspect_indices)
    logits = _tc3(gs[:6], gs[6][:, :1], b2r, wcp, bcp)
    return logits[:, :bc.shape[0]]


# speedup vs baseline: 8.2999x; 1.0141x over previous
import functools

import jax
import jax.numpy as jnp
from jax import lax
from jax.experimental import pallas as pl
from jax.experimental.pallas import tpu as pltpu
from jax.experimental.pallas import tpu_sc as plsc

N = 10000
NP = 10240
E = 320000
EP = 327680
D = 128
DH = 64
A = 1024
NC = 2
NS = 16
NW = NC * NS
ET = EP // NW
CH = 128
NCH = ET // CH
RPT = NP // NS
L = 16

_f32 = jnp.float32
_i32 = jnp.int32
_mesh = plsc.VectorSubcoreMesh(core_axis_name="c", subcore_axis_name="s")
_sc_params = pltpu.CompilerParams(use_tc_tiling_on_sc=False)
_MASK = (1 << 14) - 1


def _unpack(pki, srci, dsti, want_src):
    def body(k, carry):
        for j in range(CH // L):
            v = pki[k, pl.ds(j * L, L)]
            if want_src:
                srci[k, pl.ds(j * L, L)] = v & _MASK
            dsti[k, pl.ds(j * L, L)] = lax.shift_right_logical(v, 14)
        return carry

    lax.fori_loop(0, NCH, body, 0)


@functools.partial(
    pl.kernel,
    out_type=(jax.ShapeDtypeStruct((NP, 16), _f32),
              jax.ShapeDtypeStruct((NP, 16), _f32)),
    mesh=_mesh,
    compiler_params=_sc_params,
    scratch_types=[
        pltpu.VMEM((NCH, CH), _i32),
        pltpu.VMEM((NCH, CH), _i32),
        pltpu.VMEM((CH, 16), _f32),
        pltpu.VMEM_SHARED((NP, 16), _f32),
    ],
)
def _deg_kernel(pk2, ones16, out0, out1, pki, dsti, ones_v, acc):
    c = lax.axis_index("c")
    s = lax.axis_index("s")
    wid = s * NC + c
    r0 = s * RPT
    pltpu.sync_copy(pk2.at[pl.ds(wid * NCH, NCH)], pki)
    pltpu.sync_copy(ones16.at[pl.ds(0, CH)], ones_v)
    pltpu.sync_copy(ones16.at[pl.ds(r0, RPT)], acc.at[pl.ds(r0, RPT)])
    _unpack(pki, None, dsti, want_src=False)
    plsc.subcore_barrier()

    def body(k, carry):
        pltpu.sync_copy(ones_v, acc.at[dsti.at[k]], add=True)
        return carry

    lax.fori_loop(0, NCH, body, 0)
    plsc.subcore_barrier()

    @pl.when(c == 0)
    def _():
        pltpu.sync_copy(acc.at[pl.ds(r0, RPT)], out0.at[pl.ds(r0, RPT)])

    @pl.when(c == 1)
    def _():
        pltpu.sync_copy(acc.at[pl.ds(r0, RPT)], out1.at[pl.ds(r0, RPT)])


@functools.partial(
    pl.kernel,
    out_type=tuple(jax.ShapeDtypeStruct((NP, DH), _f32) for _ in range(4)),
    mesh=_mesh,
    compiler_params=_sc_params,
    scratch_types=[
        pltpu.VMEM((NCH, CH), _i32),
        pltpu.VMEM((NCH, CH), _i32),
        pltpu.VMEM((NCH, CH), _i32),
        pltpu.VMEM((CH, DH), _f32),
        pltpu.VMEM((CH, DH), _f32),
        pltpu.VMEM_SHARED((NP, DH), _f32),
        pltpu.SemaphoreType.DMA,
        pltpu.SemaphoreType.DMA,
    ],
)
def _edge_kernel(ylo, yhi, pk2, zc, o0lo, o0hi, o1lo, o1hi,
                 pki, srci, dsti, rows_a, rows_b, acc, sem_a, sem_b):
    c = lax.axis_index("c")
    s = lax.axis_index("s")
    wid = s * NC + c
    r0 = s * RPT
    pltpu.sync_copy(pk2.at[pl.ds(wid * NCH, NCH)], pki)
    _unpack(pki, srci, dsti, want_src=True)

    for y, out0, out1 in ((ylo, o0lo, o1lo), (yhi, o0hi, o1hi)):
        pltpu.sync_copy(zc.at[pl.ds(r0, RPT)], acc.at[pl.ds(r0, RPT)])
        plsc.subcore_barrier()

        pltpu.async_copy(y.at[srci.at[0]], rows_a, sem_a)

        def body(i, carry, y=y):
            k = 2 * i
            pltpu.async_copy(y.at[srci.at[k + 1]], rows_b, sem_b)
            pltpu.make_async_copy(y.at[srci.at[k]], rows_a, sem_a).wait()
            pltpu.sync_copy(rows_a, acc.at[dsti.at[k]], add=True)

            @pl.when(k + 2 < NCH)
            def _():
                pltpu.async_copy(y.at[srci.at[k + 2]], rows_a, sem_a)

            pltpu.make_async_copy(y.at[srci.at[k + 1]], rows_b, sem_b).wait()
            pltpu.sync_copy(rows_b, acc.at[dsti.at[k + 1]], add=True)
            return carry

        lax.fori_loop(0, NCH // 2, body, 0)
        plsc.subcore_barrier()

        @pl.when(c == 0)
        def _():
            pltpu.sync_copy(acc.at[pl.ds(r0, RPT)], out0.at[pl.ds(r0, RPT)])

        @pl.when(c == 1)
        def _():
            pltpu.sync_copy(acc.at[pl.ds(r0, RPT)], out1.at[pl.ds(r0, RPT)])

        plsc.subcore_barrier()


_APT = A // NW


@functools.partial(
    pl.kernel,
    out_type=tuple(jax.ShapeDtypeStruct((A, DH), _f32) for _ in range(6))
    + (jax.ShapeDtypeStruct((A, 16), _f32),),
    mesh=_mesh,
    compiler_params=_sc_params,
    scratch_types=[
        pltpu.VMEM((_APT,), _i32),
        pltpu.VMEM((_APT, DH), _f32),
        pltpu.VMEM((_APT, 16), _f32),
        pltpu.SemaphoreType.DMA,
    ],
)
def _gather_kernel(t0, t1, t2, t3, t4, t5, di16, asp,
                   g0, g1, g2, g3, g4, g5, gd,
                   idxv, rbuf, rbuf16, sem):
    c = lax.axis_index("c")
    s = lax.axis_index("s")
    wid = s * NC + c
    base = wid * _APT
    pltpu.sync_copy(asp.at[pl.ds(base, _APT)], idxv)

    for tab, g in ((t0, g0), (t1, g1), (t2, g2), (t3, g3), (t4, g4), (t5, g5)):
        pltpu.async_copy(tab.at[idxv], rbuf, sem).wait()
        pltpu.sync_copy(rbuf, g.at[pl.ds(base, _APT)])

    pltpu.async_copy(di16.at[idxv], rbuf16, sem).wait()
    pltpu.sync_copy(rbuf16, gd.at[pl.ds(base, _APT)])


_RB = 1024


def _tc1_body(d0, d1, x, w, ylo_ref, yhi_ref, di_ref, di16_ref):
    deg = d0[...] + d1[...] - 1.0
    di = jnp.where(deg > 0, lax.rsqrt(jnp.maximum(deg, 1e-12)), 0.0)
    y = jnp.dot(x[...], w[...], preferred_element_type=_f32) * di
    ylo_ref[...] = y[:, :DH]
    yhi_ref[...] = y[:, DH:]
    di_ref[...] = di
    di16_ref[...] = jnp.broadcast_to(di, (di.shape[0], 16))


def _tc1(d0s, d1s, x, w):
    return pl.pallas_call(
        _tc1_body,
        grid=(NP // _RB,),
        in_specs=[
            pl.BlockSpec((_RB, 1), lambda i: (i, 0)),
            pl.BlockSpec((_RB, 1), lambda i: (i, 0)),
            pl.BlockSpec((_RB, D), lambda i: (i, 0)),
            pl.BlockSpec((D, D), lambda i: (0, 0)),
        ],
        out_specs=[
            pl.BlockSpec((_RB, DH), lambda i: (i, 0)),
            pl.BlockSpec((_RB, DH), lambda i: (i, 0)),
            pl.BlockSpec((_RB, 1), lambda i: (i, 0)),
            pl.BlockSpec((_RB, 16), lambda i: (i, 0)),
        ],
        out_shape=[
            jax.ShapeDtypeStruct((NP, DH), _f32),
            jax.ShapeDtypeStruct((NP, DH), _f32),
            jax.ShapeDtypeStruct((NP, 1), _f32),
            jax.ShapeDtypeStruct((NP, 16), _f32),
        ],
    )(d0s, d1s, x, w)


def _tc2_body(a0lo, a0hi, a1lo, a1hi, ylo, yhi, di, b1, w2,
              y2lo_ref, y2hi_ref):
    dv = di[...]
    hlo = a0lo[...] + a1lo[...] + ylo[...]
    hhi = a0hi[...] + a1hi[...] + yhi[...]
    h = dv * jnp.concatenate([hlo, hhi], axis=1) + b1[...]
    h = jnp.maximum(h, 0.0)
    y2 = jnp.dot(h, w2[...], preferred_element_type=_f32) * dv
    y2lo_ref[...] = y2[:, :DH]
    y2hi_ref[...] = y2[:, DH:]


def _tc2(a0lo, a0hi, a1lo, a1hi, ylo, yhi, di, b1r, w2):
    blk = pl.BlockSpec((_RB, DH), lambda i: (i, 0))
    return pl.pallas_call(
        _tc2_body,
        grid=(NP // _RB,),
        in_specs=[
            blk, blk, blk, blk, blk, blk,
            pl.BlockSpec((_RB, 1), lambda i: (i, 0)),
            pl.BlockSpec((1, D), lambda i: (0, 0)),
            pl.BlockSpec((D, D), lambda i: (0, 0)),
        ],
        out_specs=[blk, blk],
        out_shape=[
            jax.ShapeDtypeStruct((NP, DH), _f32),
            jax.ShapeDtypeStruct((NP, DH), _f32),
        ],
    )(a0lo, a0hi, a1lo, a1hi, ylo, yhi, di, b1r, w2)


def _tc3_body(g0, g1, g2, g3, g4, g5, di, b2, wc, bc, out_ref):
    hlo = g0[...] + g2[...] + g4[...]
    hhi = g1[...] + g3[...] + g5[...]
    h = di[...] * jnp.concatenate([hlo, hhi], axis=1) + b2[...]
    out_ref[...] = jnp.dot(h, wc[...], preferred_element_type=_f32) + bc[...]


def _tc3(gs, gd, b2r, wcp, bcp):
    blk = pl.BlockSpec((A, DH), lambda i: (0, 0))
    return pl.pallas_call(
        _tc3_body,
        grid=(1,),
        in_specs=[
            blk, blk, blk, blk, blk, blk,
            pl.BlockSpec((A, 1), lambda i: (0, 0)),
            pl.BlockSpec((1, D), lambda i: (0, 0)),
            pl.BlockSpec((D, D), lambda i: (0, 0)),
            pl.BlockSpec((1, D), lambda i: (0, 0)),
        ],
        out_specs=pl.BlockSpec((A, D), lambda i: (0, 0)),
        out_shape=jax.ShapeDtypeStruct((A, D), _f32),
    )(*gs, gd, b2r, wcp, bcp)


@jax.jit
def kernel(features, edge_index, aspect_indices, W1, b1, W2, b2, Wc, bc):
    pad_src = jnp.zeros((EP - E,), edge_index.dtype)
    pad_dst = jnp.full((EP - E,), N, edge_index.dtype)
    src = jnp.concatenate([edge_index[0], pad_src])
    dst = jnp.concatenate([edge_index[1], pad_dst])
    pk2 = (src | (dst << 14)).reshape(EP // CH, CH)
    xp = jnp.pad(features, ((0, NP - N), (0, 0)))
    ones16 = jnp.ones((NP, 16), _f32)
    zc = jnp.zeros((NP, DH), _f32)
    b1r = b1.reshape(1, D)
    b2r = b2.reshape(1, D)
    wcp = jnp.pad(Wc, ((0, 0), (0, D - Wc.shape[1])))
    bcp = jnp.pad(bc, (0, D - bc.shape[0])).reshape(1, D)

    d0, d1 = _deg_kernel(pk2, ones16)
    y1lo, y1hi, dinv, di16 = _tc1(d0[:, :1], d1[:, :1], xp, W1)
    a0lo, a0hi, a1lo, a1hi = _edge_kernel(y1lo, y1hi, pk2, zc)
    y2lo, y2hi = _tc2(a0lo, a0hi, a1lo, a1hi, y1lo, y1hi, dinv, b1r, W2)
    c0lo, c0hi, c1lo, c1hi = _edge_kernel(y2lo, y2hi, pk2, zc)
    gs = _gather_kernel(c0lo, c0hi, c1lo, c1hi, y2lo, y2hi,
                        di16, aspect_indices)
    logits = _tc3(gs[:6], gs[6][:, :1], b2r, wcp, bcp)
    return logits[:, :bc.shape[0]]
